# Initial kernel scaffold; baseline (speedup 1.0000x reference)
import jax
import jax.numpy as jnp
from jax.experimental import pallas as pl


def kernel(node_attrs, edge_attrs, edge_indices, node_indices, edge_batch_indices, nodes_per_graph, tokens, concept_vocabulary, property_embeddings, tag_default, tag_W, lstm_Wih, lstm_Whh, lstm_bih, lstm_bhh, dec_Wih, dec_Whh, dec_bih, dec_bhh, W_np, W_edge, w_node_score, w_rel_score, lin_W, lin_b):
    raise NotImplementedError("write your pallas kernel here")


# speedup vs baseline: 16.9675x; 16.9675x over previous
import jax
import jax.numpy as jnp
from jax.experimental import pallas as pl
from jax.experimental.pallas import tpu as pltpu

_NI = 8


def _dot(a, b, dims):
    return jax.lax.dot_general(a, b, (dims, ((), ())),
                               preferred_element_type=jnp.float32)


def _elu(x):
    return jnp.where(x > 0, x, jnp.exp(jnp.minimum(x, 0.0)) - 1.0)


def _rsoftmax(x):
    m = jnp.max(x, axis=-1, keepdims=True)
    e = jnp.exp(x - m)
    return e / jnp.sum(e, axis=-1, keepdims=True)


def _tagger_kernel(tok_ref, voc_ref, w_ref, td_ref, out_ref):
    T = tok_ref[0]
    Tw = _dot(T, w_ref[...], ((1,), (0,)))
    voc = voc_ref[...]
    l1 = _dot(Tw, voc, ((1,), (1,)))
    l2 = jnp.sum(Tw * td_ref[...], axis=1, keepdims=True)
    m = jnp.maximum(jnp.max(l1, axis=1, keepdims=True), l2)
    e1 = jnp.exp(l1 - m)
    e2 = jnp.exp(l2 - m)
    z = jnp.sum(e1, axis=1, keepdims=True) + e2
    out_ref[0] = (e2 / z) * T + _dot(e1 / z, voc, ((1,), (0,)))


def _tagger(tokens, voc, tag_W, tag_default):
    B, L, D = tokens.shape
    V = voc.shape[0]
    return pl.pallas_call(
        _tagger_kernel,
        grid=(B,),
        in_specs=[
            pl.BlockSpec((1, L, D), lambda b: (b, 0, 0)),
            pl.BlockSpec((V, D), lambda b: (0, 0)),
            pl.BlockSpec((D, D), lambda b: (0, 0)),
            pl.BlockSpec((1, D), lambda b: (0, 0)),
        ],
        out_specs=pl.BlockSpec((1, L, D), lambda b: (b, 0, 0)),
        out_shape=jax.ShapeDtypeStruct((B, L, D), jnp.float32),
        compiler_params=pltpu.CompilerParams(
            dimension_semantics=("parallel",)),
    )(tokens, voc, tag_W, tag_default.reshape(1, D))


def _encoder_kernel(tg_ref, Wih_ref, Whh_ref, bih_ref, bhh_ref,
                    dWih_ref, dWhh_ref, dbih_ref, dbhh_ref, pe_ref,
                    enc_ref, ins_ref, foo_ref):
    tagged = tg_ref[...]
    B, L, D = tagged.shape
    Wih = Wih_ref[...]
    Whh = Whh_ref[...]
    h = jnp.zeros((B, D), jnp.float32)
    c = jnp.zeros((B, D), jnp.float32)
    for t in range(L):
        x = tagged[:, t, :]
        g = (_dot(x, Wih, ((1,), (1,))) + bih_ref[...] +
             _dot(h, Whh, ((1,), (1,))) + bhh_ref[...])
        i = g[:, :D]
        f = g[:, D:2 * D]
        gg = g[:, 2 * D:3 * D]
        o = g[:, 3 * D:]
        c = jax.nn.sigmoid(f) * c + jax.nn.sigmoid(i) * jnp.tanh(gg)
        h = jax.nn.sigmoid(o) * jnp.tanh(c)
    enc_ref[...] = h
    dWih = dWih_ref[...]
    dWhh = dWhh_ref[...]
    pe = pe_ref[...]
    hx = jnp.zeros((B, D), jnp.float32)
    for t in range(_NI):
        hx = jax.nn.relu(_dot(h, dWih, ((1,), (1,))) + dbih_ref[...] +
                         _dot(hx, dWhh, ((1,), (1,))) + dbhh_ref[...])
        scores = jnp.sum(hx[:, None, :] * tagged, axis=2)
        w = _rsoftmax(scores)
        instr = jnp.sum(w[:, :, None] * tagged, axis=1)
        ins_ref[t] = instr
        foo_ref[t] = _rsoftmax(_dot(instr, pe, ((1,), (1,))))


def _encoder(tagged, Wih, Whh, bih, bhh, dWih, dWhh, dbih, dbhh, pe):
    B, L, D = tagged.shape
    P1 = pe.shape[0]
    return pl.pallas_call(
        _encoder_kernel,
        out_shape=[
            jax.ShapeDtypeStruct((B, D), jnp.float32),
            jax.ShapeDtypeStruct((_NI, B, D), jnp.float32),
            jax.ShapeDtypeStruct((_NI, B, P1), jnp.float32),
        ],
    )(tagged, Wih, Whh, bih.reshape(1, -1), bhh.reshape(1, -1),
      dWih, dWhh, dbih.reshape(1, -1), dbhh.reshape(1, -1), pe)


def _edges_kernel(ea_ref, we_ref, ins_ref, wrel_ref, ev_ref):
    ea = ea_ref[0]
    ex = _dot(ea, we_ref[...], ((1,), (1,)))
    ins = ins_ref[0]
    wrel = wrel_ref[...]
    rows = []
    for t in range(_NI):
        s = _elu(ex * ins[t])
        rows.append(_dot(wrel, s, ((1,), (1,))))
    ev_ref[0] = jnp.concatenate(rows, axis=0)


def _edges(ea3, W_edge, instrs_g, wrel):
    B, EPG, D = ea3.shape
    return pl.pallas_call(
        _edges_kernel,
        grid=(B,),
        in_specs=[
            pl.BlockSpec((1, EPG, D), lambda b: (b, 0, 0)),
            pl.BlockSpec((D, D), lambda b: (0, 0)),
            pl.BlockSpec((1, _NI, D), lambda b: (b, 0, 0)),
            pl.BlockSpec((1, D), lambda b: (0, 0)),
        ],
        out_specs=pl.BlockSpec((1, _NI, EPG), lambda b: (b, 0, 0)),
        out_shape=jax.ShapeDtypeStruct((B, _NI, EPG), jnp.float32),
        compiler_params=pltpu.CompilerParams(
            dimension_semantics=("parallel",)),
    )(ea3, W_edge, instrs_g, wrel)


def _nodes_kernel(na_ref, wnp_ref, ins_ref, foo_ref, wnode_ref,
                  nsv_ref, anp_ref):
    na = na_ref[0]
    NPG, P, D = na.shape
    wx = [_dot(na[:, p, :], wnp_ref[p], ((1,), (1,))) for p in range(P)]
    ins = ins_ref[0]
    fool = foo_ref[0]
    wnode = wnode_ref[...]
    rows = []
    for t in range(_NI):
        m = wx[0] * fool[t, 0]
        for p in range(1, P):
            m = m + wx[p] * fool[t, p]
        s = _elu(ins[t] * m)
        rows.append(_dot(wnode, s, ((1,), (1,))))
    nsv_ref[0] = jnp.concatenate(rows, axis=0)
    anp = na[:, 0, :] * fool[_NI - 1, 0]
    for p in range(1, P):
        anp = anp + na[:, p, :] * fool[_NI - 1, p]
    anp_ref[0] = anp


def _nodes(na4, W_np, instrs_g, foo_g, wnode):
    B, NPG, P, D = na4.shape
    P1 = foo_g.shape[2]
    return pl.pallas_call(
        _nodes_kernel,
        grid=(B,),
        in_specs=[
            pl.BlockSpec((1, NPG, P, D), lambda b: (b, 0, 0, 0)),
            pl.BlockSpec((P, D, D), lambda b: (0, 0, 0)),
            pl.BlockSpec((1, _NI, D), lambda b: (b, 0, 0)),
            pl.BlockSpec((1, _NI, P1), lambda b: (b, 0, 0)),
            pl.BlockSpec((1, D), lambda b: (0, 0)),
        ],
        out_specs=[
            pl.BlockSpec((1, _NI, NPG), lambda b: (b, 0, 0)),
            pl.BlockSpec((1, NPG, D), lambda b: (b, 0, 0)),
        ],
        out_shape=[
            jax.ShapeDtypeStruct((B, _NI, NPG), jnp.float32),
            jax.ShapeDtypeStruct((B, NPG, D), jnp.float32),
        ],
        compiler_params=pltpu.CompilerParams(
            dimension_semantics=("parallel",)),
    )(na4, W_np, instrs_g, foo_g, wnode)


def _seq_kernel(src_ref, dst_ref, ev_ref, nsv_ref, rels_ref, npg_ref,
                anp_ref, out_ref, *, NPG):
    g = pl.program_id(0)
    base = g * NPG
    srcl = src_ref[0] - base
    dstl = dst_ref[0] - base
    EPG = srcl.shape[1]
    iota = jax.lax.broadcasted_iota(jnp.int32, (NPG, EPG), 0)
    OsT = (iota == srcl).astype(jnp.float32)
    OdT = (iota == dstl).astype(jnp.float32)
    evb = ev_ref[0]
    nsvb = nsv_ref[0]
    dist = jnp.ones((1, NPG), jnp.float32) * (1.0 / npg_ref[0, 0, 0])
    for t in range(_NI):
        ps = _rsoftmax(nsvb[t:t + 1, :])
        de = _dot(dist, OsT, ((1,), (0,)))
        w = evb[t:t + 1, :] * de
        agg = _dot(w, OdT, ((1,), (1,)))
        pr = _rsoftmax(agg)
        r = rels_ref[0, 0, t]
        dist = r * pr + (1.0 - r) * ps
    out_ref[0] = _dot(dist, anp_ref[0], ((1,), (0,)))


def _seqloop(src3, dst3, ev, nsv, rels, npg3, anp):
    B, NI, EPG = ev.shape
    NPG = nsv.shape[2]
    D = anp.shape[2]
    import functools
    return pl.pallas_call(
        functools.partial(_seq_kernel, NPG=NPG),
        grid=(B,),
        in_specs=[
            pl.BlockSpec((1, 1, EPG), lambda b: (b, 0, 0)),
            pl.BlockSpec((1, 1, EPG), lambda b: (b, 0, 0)),
            pl.BlockSpec((1, NI, EPG), lambda b: (b, 0, 0)),
            pl.BlockSpec((1, NI, NPG), lambda b: (b, 0, 0)),
            pl.BlockSpec((1, 1, NI), lambda b: (b, 0, 0)),
            pl.BlockSpec((1, 1, 1), lambda b: (b, 0, 0)),
            pl.BlockSpec((1, NPG, D), lambda b: (b, 0, 0)),
        ],
        out_specs=pl.BlockSpec((1, 1, D), lambda b: (b, 0, 0)),
        out_shape=jax.ShapeDtypeStruct((B, 1, D), jnp.float32),
        compiler_params=pltpu.CompilerParams(
            dimension_semantics=("parallel",)),
    )(src3, dst3, ev, nsv, rels, npg3, anp)


def _final_kernel(enc_ref, agg_ref, lw_ref, lb_ref, out_ref):
    D = enc_ref.shape[1]
    lw = lw_ref[...]
    out_ref[...] = (_dot(enc_ref[...], lw[:, :D], ((1,), (1,))) +
                    _dot(agg_ref[...], lw[:, D:], ((1,), (1,))) +
                    lb_ref[...])


def _final(encoded, aggregated, lin_W, lin_b):
    B = encoded.shape[0]
    OUT = lin_W.shape[0]
    return pl.pallas_call(
        _final_kernel,
        out_shape=jax.ShapeDtypeStruct((B, OUT), jnp.float32),
    )(encoded, aggregated, lin_W, lin_b.reshape(1, -1))


def kernel(node_attrs, edge_attrs, edge_indices, node_indices,
           edge_batch_indices, nodes_per_graph, tokens, concept_vocabulary,
           property_embeddings, tag_default, tag_W, lstm_Wih, lstm_Whh,
           lstm_bih, lstm_bhh, dec_Wih, dec_Whh, dec_bih, dec_bhh,
           W_np, W_edge, w_node_score, w_rel_score, lin_W, lin_b):
    B, L, D = tokens.shape
    N, P, _ = node_attrs.shape
    E = edge_attrs.shape[0]
    NPG = N // B
    EPG = E // B

    tagged = _tagger(tokens, concept_vocabulary, tag_W, tag_default)
    encoded, instrs, foo = _encoder(
        tagged, lstm_Wih, lstm_Whh, lstm_bih, lstm_bhh,
        dec_Wih, dec_Whh, dec_bih, dec_bhh, property_embeddings)
    instrs_g = jnp.transpose(instrs, (1, 0, 2))
    foo_g = jnp.transpose(foo, (1, 0, 2))

    ev = _edges(edge_attrs.reshape(B, EPG, D), W_edge, instrs_g,
                w_rel_score.reshape(1, D))
    nsv, anp = _nodes(node_attrs.reshape(B, NPG, P, D), W_np, instrs_g,
                      foo_g, w_node_score.reshape(1, D))

    rels = foo_g[:, :, P].reshape(B, 1, _NI)
    src3 = edge_indices[0].reshape(B, 1, EPG)
    dst3 = edge_indices[1].reshape(B, 1, EPG)
    aggr = _seqloop(src3, dst3, ev, nsv, rels,
                    nodes_per_graph.reshape(B, 1, 1), anp)

    return _final(encoded, aggr.reshape(B, D), lin_W, lin_b)
